# baseline (speedup 1.0000x reference)
import functools

import jax
import jax.numpy as jnp
from jax import lax
from jax.experimental import pallas as pl
from jax.experimental.pallas import tpu as pltpu
from jax.experimental.pallas import tpu_sc as plsc

M = 100000
B = 16384
D = 128
NW = 32
S = 3136
S_LAST = M - S * (NW - 1)
T = S
CH = 128
WB = S + CH
CPC = 224
NSEG = 12
TAIL_LAST = S_LAST - NSEG * CPC
NCHUNK = B // 16
DPS = 85
BF = 8


def _dedup_chunks(idx_v, table_v, row_lo, n_own, iota, base, chunks):
  for group_start in range(0, len(chunks), BF):
    group = chunks[group_start:group_start + BF]
    ivecs = [idx_v[pl.ds((base + c) * 16, 16)] for c in group]
    locals_ = [ivec - row_lo for ivec in ivecs]
    valids = [(l >= 0) & (l < n_own) for l in locals_]
    lasts = [plsc.scan_count(ivec, mask=v)[1]
             for ivec, v in zip(ivecs, valids)]
    for cc, l, v, last in zip(group, locals_, valids, lasts):
      m = v & last
      l_c = jnp.clip(l, 0, T - 1)
      plsc.store_scatter(table_v, [l_c], (base + cc) * 16 + iota, mask=m)


def _body(idx_hbm, val_hbm, mem_hbm, out_hbm, idx_v, table_v, nodes_v,
          gidx_v, nchunk_n, rows_v, cbuf0, cbuf1, gsem0, gsem1, ssem0,
          ssem1, dma_sem):
  c = lax.axis_index("c")
  s = lax.axis_index("s")
  wid = s * 2 + c
  row_lo = wid * S
  n_own = jnp.where(wid == NW - 1, S_LAST, S)
  cbufs = (cbuf0, cbuf1)
  gsems = (gsem0, gsem1)
  ssems = (ssem0, ssem1)
  iota = lax.iota(jnp.int32, 16)

  def gather_cp(k, off):
    return pltpu.make_async_copy(
        mem_hbm.at[pl.ds(off, CPC)], cbufs[k % 2], gsems[k % 2])

  def scatter_cp(k, off):
    return pltpu.make_async_copy(
        cbufs[k % 2], out_hbm.at[pl.ds(off, CPC)], ssems[k % 2])

  pltpu.sync_copy(idx_hbm, idx_v)

  minus1 = jnp.full((16,), -1, jnp.int32)

  def zero_body(i, carry):
    for u in range(4):
      table_v[pl.ds((i * 4 + u) * 16, 16)] = minus1
    return carry

  lax.fori_loop(0, T // 16 // 4, zero_body, 0)

  def seg_body(i, carry):
    off = row_lo + i * CPC
    for par in range(2):
      @pl.when((i & 1) == par)
      def _():
        @pl.when(i >= 2)
        def _():
          scatter_cp(par, off - 2 * CPC).wait()
        gather_cp(par, off).start()

    _dedup_chunks(idx_v, table_v, row_lo, n_own, iota, i * DPS,
                  list(range(DPS)))

    for par in range(2):
      @pl.when((i & 1) == par)
      def _():
        gather_cp(par, off).wait()
        scatter_cp(par, off).start()
    return carry

  lax.fori_loop(0, NSEG, seg_body, 0)

  _dedup_chunks(idx_v, table_v, row_lo, n_own, iota, NSEG * DPS,
                list(range(NCHUNK - NSEG * DPS)))

  @pl.when(wid < NW - 1)
  def _():
    for k in (NSEG, NSEG + 1):
      off = row_lo + k * CPC
      scatter_cp(k, off - 2 * CPC).wait()
      gather_cp(k, off).start()
      gather_cp(k, off).wait()
      scatter_cp(k, off).start()
    scatter_cp(NSEG, row_lo + NSEG * CPC).wait()
    scatter_cp(NSEG + 1, row_lo + (NSEG + 1) * CPC).wait()

  @pl.when(wid == NW - 1)
  def _():
    scatter_cp(0, row_lo + (NSEG - 2) * CPC).wait()
    scatter_cp(1, row_lo + (NSEG - 1) * CPC).wait()
    off = row_lo + NSEG * CPC
    pltpu.make_async_copy(mem_hbm.at[pl.ds(off, TAIL_LAST)],
                          cbuf0.at[pl.ds(0, TAIL_LAST)], gsem0).start()
    pltpu.make_async_copy(mem_hbm.at[pl.ds(off, TAIL_LAST)],
                          cbuf0.at[pl.ds(0, TAIL_LAST)], gsem0).wait()
    pltpu.make_async_copy(cbuf0.at[pl.ds(0, TAIL_LAST)],
                          out_hbm.at[pl.ds(off, TAIL_LAST)], ssem0).start()
    pltpu.make_async_copy(cbuf0.at[pl.ds(0, TAIL_LAST)],
                          out_hbm.at[pl.ds(off, TAIL_LAST)], ssem0).wait()

  def extract_body(t, off):
    tv = table_v[pl.ds(t * 16, 16)]
    m = tv >= 0
    nodes = (row_lo + t * 16) + iota
    plsc.store_compressed(nodes_v.at[pl.ds(off, 16)], nodes, mask=m)
    plsc.store_compressed(gidx_v.at[pl.ds(off, 16)], tv, mask=m)
    return off + jnp.sum(m.astype(jnp.int32))

  nwin = lax.fori_loop(0, T // 16, extract_body, jnp.int32(0))

  @pl.when(nwin > 0)
  def _():
    lane0 = (iota == 0).astype(jnp.int32)
    n0 = jnp.sum(nodes_v[pl.ds(0, 16)] * lane0)
    g0 = jnp.sum(gidx_v[pl.ds(0, 16)] * lane0)
    npad = jnp.zeros((16,), jnp.int32) + n0
    gpad = jnp.zeros((16,), jnp.int32) + g0
    for k in range(CH // 16):
      nodes_v[pl.ds(nwin + k * 16, 16)] = npad
      gidx_v[pl.ds(nwin + k * 16, 16)] = gpad

  nchunks = (nwin + CH - 1) // CH

  def chunk_body(ci, carry):
    off = ci * CH
    for k in range(CH // 16):
      nchunk_n[pl.ds(k * 16, 16)] = nodes_v[pl.ds(off + k * 16, 16)]
    pltpu.async_copy(val_hbm.at[gidx_v.at[pl.ds(off, CH)]], rows_v,
                     dma_sem).wait()
    pltpu.async_copy(rows_v, out_hbm.at[nchunk_n], dma_sem).wait()
    return carry

  lax.fori_loop(0, nchunks, chunk_body, 0)


_agg = functools.partial(
    pl.kernel,
    out_type=jax.ShapeDtypeStruct((M, D), jnp.float32),
    mesh=plsc.VectorSubcoreMesh(core_axis_name="c", subcore_axis_name="s"),
    compiler_params=pltpu.CompilerParams(needs_layout_passes=False),
    scratch_types=[
        pltpu.VMEM((B,), jnp.int32),
        pltpu.VMEM((T,), jnp.int32),
        pltpu.VMEM((WB,), jnp.int32),
        pltpu.VMEM((WB,), jnp.int32),
        pltpu.VMEM((CH,), jnp.int32),
        pltpu.VMEM((CH, D), jnp.float32),
        pltpu.VMEM((CPC, D), jnp.float32),
        pltpu.VMEM((CPC, D), jnp.float32),
        pltpu.SemaphoreType.DMA,
        pltpu.SemaphoreType.DMA,
        pltpu.SemaphoreType.DMA,
        pltpu.SemaphoreType.DMA,
        pltpu.SemaphoreType.DMA,
    ],
)(_body)


def kernel(mem, idx, val):
  idx32 = idx.astype(jnp.int32)
  return _agg(idx32, val, mem)
